# baseline (speedup 1.0000x reference)
import functools

import jax
import jax.numpy as jnp
import numpy as np
from jax.experimental import pallas as pl
from jax.experimental.pallas import tpu as pltpu
from jax.experimental.pallas import tpu_sc as plsc

_COORDS_XY = np.array(
    [[2, 1], [4, 1], [1, 2], [2, 2], [3, 2], [4, 2], [5, 2], [1, 3], [2, 3],
     [3, 3], [4, 3], [5, 3], [1, 4], [2, 4], [3, 4], [4, 4], [5, 4], [2, 5],
     [4, 5]], dtype=np.int32)

_N = 19
_HALF = 1024


@functools.partial(
    pl.kernel,
    mesh=plsc.ScalarSubcoreMesh(axis_name="c", num_cores=1),
    out_type=jax.ShapeDtypeStruct((2 * _N, _HALF), jnp.float32),
    scratch_types=[
        pltpu.VMEM_SHARED((5, _HALF), jnp.float32),
        pltpu.SemaphoreType.DMA,
    ],
)
def _pe_gather(table_hbm, out_hbm, rows_spm, sem):
    pltpu.sync_copy(table_hbm.at[pl.ds(1, 5)], rows_spm)
    copies = []
    for i in range(_N):
        for j in range(2):
            c = int(_COORDS_XY[i, j])
            copies.append(pltpu.async_copy(
                rows_spm.at[pl.ds(c - 1, 1)],
                out_hbm.at[pl.ds(2 * i + j, 1)], sem))
    for c in copies:
        c.wait()


def kernel(x, pe):
    del x
    table = pe.reshape(pe.shape[1], pe.shape[2])
    out = _pe_gather(table)
    return out.reshape(1, _N, 2 * _HALF)
